# baseline (speedup 1.0000x reference)
import functools

import jax
import jax.numpy as jnp
from jax import lax
from jax.experimental import pallas as pl
from jax.experimental.pallas import tpu as pltpu
from jax.experimental.pallas import tpu_sc as plsc

_VOCAB = 1000
_EMBED = 64
_BATCH = 16384

_NC = 2
_NS = 16
_NW = _NC * _NS
_BPW = _BATCH // _NW
_CHUNK = 32
_NCHUNK = _BPW // _CHUNK


def _table_body(e_ref, w_ref, b_ref, o_ref):
    o_ref[...] = lax.dot_general(
        e_ref[...], w_ref[...],
        dimension_numbers=(((1,), (1,)), ((), ())),
        preferred_element_type=jnp.float32,
    ) + b_ref[...]


def _make_table(emb_table, W, b):
    return pl.pallas_call(
        _table_body,
        out_shape=jax.ShapeDtypeStruct((_VOCAB, _VOCAB), jnp.float32),
    )(emb_table, W, b.reshape(1, _VOCAB))


_sc_mesh = plsc.VectorSubcoreMesh(core_axis_name="c", subcore_axis_name="s")


@functools.partial(
    pl.kernel,
    mesh=_sc_mesh,
    out_type=jax.ShapeDtypeStruct((_BATCH, _VOCAB), jnp.float32),
    scratch_types=[
        pltpu.VMEM((_BPW,), jnp.int32),
        pltpu.VMEM((2, _CHUNK, _VOCAB), jnp.float32),
        pltpu.SemaphoreType.DMA,
        pltpu.SemaphoreType.DMA,
        pltpu.SemaphoreType.DMA,
        pltpu.SemaphoreType.DMA,
    ],
    compiler_params=pltpu.CompilerParams(use_tc_tiling_on_sc=False),
)
def _sc_gather(table_hbm, idx_hbm, out_hbm, idx_v, rows_v,
               gsem0, gsem1, ssem0, ssem1):
    wid = lax.axis_index("s") * _NC + lax.axis_index("c")
    base = wid * _BPW
    pltpu.sync_copy(idx_hbm.at[pl.ds(base, _BPW)], idx_v)

    gsems = (gsem0, gsem1)
    ssems = (ssem0, ssem1)

    def start_gather(g):
        return pltpu.async_copy(
            table_hbm.at[idx_v.at[pl.ds(g * _CHUNK, _CHUNK)]],
            rows_v.at[g % 2],
            gsems[g % 2],
        )

    def start_store(g):
        return pltpu.async_copy(
            rows_v.at[g % 2],
            out_hbm.at[pl.ds(base + g * _CHUNK, _CHUNK)],
            ssems[g % 2],
        )

    gathers = [start_gather(0)]
    stores = [None, None]
    for g in range(_NCHUNK):
        nxt = g + 1
        if nxt < _NCHUNK:
            if stores[nxt % 2] is not None:
                stores[nxt % 2].wait()
                stores[nxt % 2] = None
            gathers.append(start_gather(nxt))
        gathers[g].wait()
        stores[g % 2] = start_store(g)
    for s in stores:
        if s is not None:
            s.wait()


def kernel(target, emb_table, W, b):
    table = _make_table(emb_table, W, b)
    return _sc_gather(table, target.astype(jnp.int32))


# speedup vs baseline: 1.4568x; 1.4568x over previous
import functools

import jax
import jax.numpy as jnp
from jax import lax
from jax.experimental import pallas as pl
from jax.experimental.pallas import tpu as pltpu
from jax.experimental.pallas import tpu_sc as plsc

_VOCAB = 1000
_VPAD = 1024
_MAIN = 896
_TAIL = _VOCAB - _MAIN
_EMBED = 64
_BATCH = 16384

_NC = 2
_NS = 16
_NW = _NC * _NS
_BPW = _BATCH // _NW
_CHUNK = 32
_NCHUNK = _BPW // _CHUNK


def _table_body(e_ref, w_ref, b_ref, o_ref):
    o_ref[...] = lax.dot_general(
        e_ref[...], w_ref[...],
        dimension_numbers=(((1,), (1,)), ((), ())),
        preferred_element_type=jnp.float32,
    ) + b_ref[...]


def _make_table(emb_table, W, b):
    w_pad = jnp.pad(W, ((0, _VPAD - _VOCAB), (0, 0)))
    b_pad = jnp.pad(b, (0, _VPAD - _VOCAB)).reshape(1, _VPAD)
    return pl.pallas_call(
        _table_body,
        out_shape=jax.ShapeDtypeStruct((_VOCAB, _VPAD), jnp.float32),
    )(emb_table, w_pad, b_pad)


_sc_mesh = plsc.VectorSubcoreMesh(core_axis_name="c", subcore_axis_name="s")


@functools.partial(
    pl.kernel,
    mesh=_sc_mesh,
    out_type=jax.ShapeDtypeStruct((_BATCH, _VOCAB), jnp.float32),
    scratch_types=[
        pltpu.VMEM((_BPW,), jnp.int32),
        pltpu.VMEM((2, _CHUNK, _VPAD), jnp.float32),
        pltpu.VMEM((2, _CHUNK, _TAIL), jnp.float32),
        pltpu.SemaphoreType.DMA,
        pltpu.SemaphoreType.DMA,
        pltpu.SemaphoreType.DMA,
        pltpu.SemaphoreType.DMA,
        pltpu.SemaphoreType.DMA,
        pltpu.SemaphoreType.DMA,
    ],
)
def _sc_gather(table_hbm, idx_hbm, out_hbm, idx_v, rows_v, tail_v,
               gsem0, gsem1, msem0, msem1, tsem0, tsem1):
    wid = lax.axis_index("s") * _NC + lax.axis_index("c")
    base = wid * _BPW
    pltpu.sync_copy(idx_hbm.at[pl.ds(base, _BPW)], idx_v)

    gsems = (gsem0, gsem1)
    msems = (msem0, msem1)
    tsems = (tsem0, tsem1)
    iota = lax.iota(jnp.int32, 16)

    def start_gather(g):
        return pltpu.async_copy(
            table_hbm.at[idx_v.at[pl.ds(g * _CHUNK, _CHUNK)]],
            rows_v.at[g % 2],
            gsems[g % 2],
        )

    def compact_tail(g):
        rows_ref = rows_v.at[g % 2]
        tail_ref = tail_v.at[g % 2]

        def body(r, carry):
            for j in range(_TAIL // 16):
                tail_ref[r, pl.ds(16 * j, 16)] = (
                    rows_ref[r, pl.ds(_MAIN + 16 * j, 16)])
            if _TAIL % 16:
                off = _TAIL - 16
                tail_ref[r, pl.ds(off, 16)] = (
                    rows_ref[r, pl.ds(_MAIN + off, 16)])
            return carry

        lax.fori_loop(0, _CHUNK, body, 0)

    def start_main_store(g):
        return pltpu.async_copy(
            rows_v.at[g % 2, :, pl.ds(0, _MAIN)],
            out_hbm.at[pl.ds(base + g * _CHUNK, _CHUNK), pl.ds(0, _MAIN)],
            msems[g % 2],
        )

    def start_tail_store(g):
        return pltpu.async_copy(
            tail_v.at[g % 2],
            out_hbm.at[pl.ds(base + g * _CHUNK, _CHUNK), pl.ds(_MAIN, _TAIL)],
            tsems[g % 2],
        )

    gathers = [start_gather(0)]
    main_stores = [None, None]
    tail_stores = [None, None]
    for g in range(_NCHUNK):
        nxt = g + 1
        if nxt < _NCHUNK:
            if main_stores[nxt % 2] is not None:
                main_stores[nxt % 2].wait()
                main_stores[nxt % 2] = None
            gathers.append(start_gather(nxt))
        gathers[g].wait()
        if tail_stores[g % 2] is not None:
            tail_stores[g % 2].wait()
            tail_stores[g % 2] = None
        compact_tail(g)
        main_stores[g % 2] = start_main_store(g)
        tail_stores[g % 2] = start_tail_store(g)
    for s in main_stores + tail_stores:
        if s is not None:
            s.wait()


def kernel(target, emb_table, W, b):
    table = _make_table(emb_table, W, b)
    return _sc_gather(table, target.astype(jnp.int32))


# speedup vs baseline: 3.0003x; 2.0595x over previous
import functools

import jax
import jax.numpy as jnp
from jax import lax
from jax.experimental import pallas as pl
from jax.experimental.pallas import tpu as pltpu
from jax.experimental.pallas import tpu_sc as plsc

_VOCAB = 1000
_EMBED = 64
_EPAD = 128
_BATCH = 16384

_NC = 2
_NS = 16
_NW = _NC * _NS
_BPW = _BATCH // _NW
_CHUNK = 128
_NCHUNK = _BPW // _CHUNK

_BBLK = 512
_NBLK = _BATCH // _BBLK


_sc_mesh = plsc.VectorSubcoreMesh(core_axis_name="c", subcore_axis_name="s")


@functools.partial(
    pl.kernel,
    mesh=_sc_mesh,
    out_type=jax.ShapeDtypeStruct((_BATCH, _EPAD), jnp.float32),
    scratch_types=[
        pltpu.VMEM((_BPW,), jnp.int32),
        pltpu.VMEM((2, _CHUNK, _EPAD), jnp.float32),
        pltpu.SemaphoreType.DMA,
        pltpu.SemaphoreType.DMA,
        pltpu.SemaphoreType.DMA,
        pltpu.SemaphoreType.DMA,
    ],
)
def _sc_embed_gather(emb_hbm, idx_hbm, out_hbm, idx_v, rows_v,
                     gsem0, gsem1, ssem0, ssem1):
    wid = lax.axis_index("s") * _NC + lax.axis_index("c")
    base = wid * _BPW
    pltpu.sync_copy(idx_hbm.at[pl.ds(base, _BPW)], idx_v)

    gsems = (gsem0, gsem1)
    ssems = (ssem0, ssem1)

    def start_gather(g):
        return pltpu.async_copy(
            emb_hbm.at[idx_v.at[pl.ds(g * _CHUNK, _CHUNK)]],
            rows_v.at[g % 2],
            gsems[g % 2],
        )

    def start_store(g):
        return pltpu.async_copy(
            rows_v.at[g % 2],
            out_hbm.at[pl.ds(base + g * _CHUNK, _CHUNK)],
            ssems[g % 2],
        )

    gathers = [start_gather(0)]
    stores = [None, None]
    for g in range(_NCHUNK):
        nxt = g + 1
        if nxt < _NCHUNK:
            if stores[nxt % 2] is not None:
                stores[nxt % 2].wait()
                stores[nxt % 2] = None
            gathers.append(start_gather(nxt))
        gathers[g].wait()
        stores[g % 2] = start_store(g)
    for s in stores:
        if s is not None:
            s.wait()


def _proj_body(x_ref, w_ref, b_ref, o_ref):
    x = x_ref[...][:, :_EMBED]
    o_ref[...] = lax.dot_general(
        w_ref[...], x,
        dimension_numbers=(((1,), (1,)), ((), ())),
        preferred_element_type=jnp.float32,
    ) + b_ref[...]


def _proj_matmul(X, W, b):
    return pl.pallas_call(
        _proj_body,
        grid=(_NBLK,),
        in_specs=[
            pl.BlockSpec((_BBLK, _EPAD), lambda j: (j, 0)),
            pl.BlockSpec((_VOCAB, _EMBED), lambda j: (0, 0)),
            pl.BlockSpec((_VOCAB, 1), lambda j: (0, 0)),
        ],
        out_specs=pl.BlockSpec((_VOCAB, _BBLK), lambda j: (0, j)),
        out_shape=jax.ShapeDtypeStruct((_VOCAB, _BATCH), jnp.float32),
    )(X, W, b.reshape(_VOCAB, 1))


def kernel(target, emb_table, W, b):
    emb_pad = jnp.pad(emb_table, ((0, 0), (0, _EPAD - _EMBED)))
    X = _sc_embed_gather(emb_pad, target.astype(jnp.int32))
    outT = _proj_matmul(X, W, b)
    return outT.T


# speedup vs baseline: 3.5371x; 1.1789x over previous
import functools

import jax
import jax.numpy as jnp
from jax import lax
from jax.experimental import pallas as pl
from jax.experimental.pallas import tpu as pltpu
from jax.experimental.pallas import tpu_sc as plsc

_VOCAB = 1000
_EMBED = 64
_EPAD = 128
_BATCH = 16384

_NC = 2
_NS = 16
_NW = _NC * _NS
_BPW = _BATCH // _NW
_CHUNK = 128
_NCHUNK = _BPW // _CHUNK

_VBLK = 200
_NBLK = _VOCAB // _VBLK


_sc_mesh = plsc.VectorSubcoreMesh(core_axis_name="c", subcore_axis_name="s")


@functools.partial(
    pl.kernel,
    mesh=_sc_mesh,
    out_type=jax.ShapeDtypeStruct((_BATCH, _EPAD), jnp.float32),
    scratch_types=[
        pltpu.VMEM((_BPW,), jnp.int32),
        pltpu.VMEM((2, _CHUNK, _EPAD), jnp.float32),
        pltpu.SemaphoreType.DMA,
        pltpu.SemaphoreType.DMA,
        pltpu.SemaphoreType.DMA,
        pltpu.SemaphoreType.DMA,
    ],
)
def _sc_embed_gather(emb_hbm, idx_hbm, out_hbm, idx_v, rows_v,
                     gsem0, gsem1, ssem0, ssem1):
    wid = lax.axis_index("s") * _NC + lax.axis_index("c")
    base = wid * _BPW
    pltpu.sync_copy(idx_hbm.at[pl.ds(base, _BPW)], idx_v)

    gsems = (gsem0, gsem1)
    ssems = (ssem0, ssem1)

    def start_gather(g):
        return pltpu.async_copy(
            emb_hbm.at[idx_v.at[pl.ds(g * _CHUNK, _CHUNK)]],
            rows_v.at[g % 2],
            gsems[g % 2],
        )

    def start_store(g):
        return pltpu.async_copy(
            rows_v.at[g % 2],
            out_hbm.at[pl.ds(base + g * _CHUNK, _CHUNK)],
            ssems[g % 2],
        )

    gathers = [start_gather(0)]
    stores = [None, None]
    for g in range(_NCHUNK):
        nxt = g + 1
        if nxt < _NCHUNK:
            if stores[nxt % 2] is not None:
                stores[nxt % 2].wait()
                stores[nxt % 2] = None
            gathers.append(start_gather(nxt))
        gathers[g].wait()
        stores[g % 2] = start_store(g)
    for s in stores:
        if s is not None:
            s.wait()


def _proj_body(x_ref, w_ref, b_ref, o_ref):
    x = x_ref[...][:, :_EMBED]
    o_ref[...] = lax.dot_general(
        w_ref[...], x,
        dimension_numbers=(((1,), (1,)), ((), ())),
        preferred_element_type=jnp.float32,
    ) + b_ref[...]


def _proj_matmul(X, W, b):
    return pl.pallas_call(
        _proj_body,
        grid=(_NBLK,),
        in_specs=[
            pl.BlockSpec((_BATCH, _EPAD), lambda j: (0, 0)),
            pl.BlockSpec((_VBLK, _EMBED), lambda j: (j, 0)),
            pl.BlockSpec((_VBLK, 1), lambda j: (j, 0)),
        ],
        out_specs=pl.BlockSpec((_VBLK, _BATCH), lambda j: (j, 0)),
        out_shape=jax.ShapeDtypeStruct((_VOCAB, _BATCH), jnp.float32),
    )(X, W, b.reshape(_VOCAB, 1))


def kernel(target, emb_table, W, b):
    emb_pad = jnp.pad(emb_table, ((0, 0), (0, _EPAD - _EMBED)))
    X = _sc_embed_gather(emb_pad, target.astype(jnp.int32))
    outT = _proj_matmul(X, W, b)
    return outT.T


# speedup vs baseline: 3.5530x; 1.0045x over previous
import functools

import jax
import jax.numpy as jnp
from jax import lax
from jax.experimental import pallas as pl
from jax.experimental.pallas import tpu as pltpu
from jax.experimental.pallas import tpu_sc as plsc

_VOCAB = 1000
_EMBED = 64
_EPAD = 128
_BATCH = 16384

_NC = 2
_NS = 16
_NW = _NC * _NS
_BPW = _BATCH // _NW
_CHUNK = 128
_NCHUNK = _BPW // _CHUNK

_VBLK = 200
_NBLK = _VOCAB // _VBLK


_sc_mesh = plsc.VectorSubcoreMesh(core_axis_name="c", subcore_axis_name="s")


@functools.partial(
    pl.kernel,
    mesh=_sc_mesh,
    out_type=jax.ShapeDtypeStruct((_BATCH, _EPAD), jnp.float32),
    scratch_types=[
        pltpu.VMEM((_BPW,), jnp.int32),
        pltpu.VMEM((_NCHUNK, _CHUNK, _EPAD), jnp.float32),
        pltpu.SemaphoreType.DMA,
        pltpu.SemaphoreType.DMA,
        pltpu.SemaphoreType.DMA,
        pltpu.SemaphoreType.DMA,
        pltpu.SemaphoreType.DMA,
        pltpu.SemaphoreType.DMA,
        pltpu.SemaphoreType.DMA,
        pltpu.SemaphoreType.DMA,
    ],
)
def _sc_embed_gather(emb_hbm, idx_hbm, out_hbm, idx_v, rows_v, *sems):
    wid = lax.axis_index("s") * _NC + lax.axis_index("c")
    base = wid * _BPW
    pltpu.sync_copy(idx_hbm.at[pl.ds(base, _BPW)], idx_v)

    gsems = sems[:_NCHUNK]
    ssems = sems[_NCHUNK:]

    gathers = [
        pltpu.async_copy(
            emb_hbm.at[idx_v.at[pl.ds(g * _CHUNK, _CHUNK)]],
            rows_v.at[g],
            gsems[g],
        )
        for g in range(_NCHUNK)
    ]
    stores = []
    for g in range(_NCHUNK):
        gathers[g].wait()
        stores.append(pltpu.async_copy(
            rows_v.at[g],
            out_hbm.at[pl.ds(base + g * _CHUNK, _CHUNK)],
            ssems[g],
        ))
    for s in stores:
        s.wait()


def _proj_body(x_ref, w_ref, b_ref, o_ref):
    o_ref[...] = lax.dot_general(
        w_ref[...], x_ref[...],
        dimension_numbers=(((1,), (1,)), ((), ())),
        preferred_element_type=jnp.float32,
    ) + b_ref[...]


def _proj_matmul(X, W, b):
    w_pad = jnp.pad(W, ((0, 0), (0, _EPAD - _EMBED)))
    return pl.pallas_call(
        _proj_body,
        grid=(_NBLK,),
        in_specs=[
            pl.BlockSpec((_BATCH, _EPAD), lambda j: (0, 0)),
            pl.BlockSpec((_VBLK, _EPAD), lambda j: (j, 0)),
            pl.BlockSpec((_VBLK, 1), lambda j: (j, 0)),
        ],
        out_specs=pl.BlockSpec((_VBLK, _BATCH), lambda j: (j, 0)),
        out_shape=jax.ShapeDtypeStruct((_VOCAB, _BATCH), jnp.float32),
    )(X, w_pad, b.reshape(_VOCAB, 1))


def kernel(target, emb_table, W, b):
    emb_pad = jnp.pad(emb_table, ((0, 0), (0, _EPAD - _EMBED)))
    X = _sc_embed_gather(emb_pad, target.astype(jnp.int32))
    outT = _proj_matmul(X, W, b)
    return outT.T
